# baseline (speedup 1.0000x reference)
import jax
import jax.numpy as jnp
from jax import lax
from jax.experimental import pallas as pl
from jax.experimental.pallas import tpu as pltpu
from jax.experimental.pallas import tpu_sc as plsc

N_NODES = 10000
N_EDGES = 320000
D = 128
NC = 2
NS = 16
NW = NC * NS
B = 128
NCHUNK = 80
EPW = NCHUNK * B
E_PAD = NW * EPW
NPAD = 10240
RPT = NPAD // NS
NB = 1


def _sc_aggregate(feat, src3d, dst3d):
    mesh = plsc.VectorSubcoreMesh(
        core_axis_name="c", subcore_axis_name="s",
        num_cores=NC, num_subcores=NS)

    def body(feat_hbm, src_hbm, dst_hbm, acc_out, cnt_out,
             src_v, dst_v, rows_v, ones_v, zc_v,
             acc_sh, cnt_sh, gsem0, gsem1, csem):
        cid = lax.axis_index("c")
        sid = lax.axis_index("s")
        wid = sid * NC + cid
        gsems = (gsem0, gsem1)

        zv = jnp.zeros((16,), jnp.float32)
        ov = jnp.ones((16,), jnp.float32)

        def zrow_loop(i, carry):
            def zcol(k, c2):
                rows_v[0, i, pl.ds(k * 16, 16)] = zv
                return c2
            return lax.fori_loop(0, D // 16, zcol, carry)
        lax.fori_loop(0, B, zrow_loop, 0)

        def zc_loop(i, carry):
            zc_v[pl.ds(i * 16, 16)] = zv
            return carry
        lax.fori_loop(0, RPT // 16, zc_loop, 0)

        def o_loop(i, carry):
            ones_v[pl.ds(i * 16, 16)] = ov
            return carry
        lax.fori_loop(0, B // 16, o_loop, 0)

        base = sid * RPT
        def zacc(r, carry):
            pltpu.sync_copy(rows_v.at[0],
                            acc_sh.at[pl.ds(base + r * B, B)])
            return carry
        lax.fori_loop(0, RPT // B, zacc, 0)
        pltpu.sync_copy(zc_v, cnt_sh.at[pl.ds(base, RPT)])

        pltpu.sync_copy(src_hbm.at[wid], src_v)
        pltpu.sync_copy(dst_hbm.at[wid], dst_v)

        plsc.subcore_barrier()

        def chunk(j, carry):
            pltpu.async_copy(
                feat_hbm.at[src_v.at[j]], rows_v.at[0], gsem0).wait()
            pltpu.sync_copy(rows_v.at[0], acc_sh.at[dst_v.at[j]], add=True)
            pltpu.sync_copy(ones_v, cnt_sh.at[dst_v.at[j]], add=True)
            return carry
        lax.fori_loop(0, NCHUNK, chunk, 0)

        plsc.subcore_barrier()

        def wb(r, carry):
            off = base + r * B
            pltpu.sync_copy(acc_sh.at[pl.ds(off, B)],
                            acc_out.at[cid].at[pl.ds(off, B)])
            return carry
        lax.fori_loop(0, RPT // B, wb, 0)
        pltpu.sync_copy(cnt_sh.at[pl.ds(base, RPT)],
                        cnt_out.at[cid].at[pl.ds(base, RPT)])

    run = pl.kernel(
        body,
        out_type=(jax.ShapeDtypeStruct((NC, NPAD, D), jnp.float32),
                  jax.ShapeDtypeStruct((NC, NPAD), jnp.float32)),
        mesh=mesh,
        compiler_params=pltpu.CompilerParams(use_tc_tiling_on_sc=False),
        scratch_types=(
            pltpu.VMEM((NCHUNK, B), jnp.int32),
            pltpu.VMEM((NCHUNK, B), jnp.int32),
            pltpu.VMEM((NB, B, D), jnp.float32),
            pltpu.VMEM((B,), jnp.float32),
            pltpu.VMEM((RPT,), jnp.float32),
            pltpu.VMEM_SHARED((NPAD, D), jnp.float32),
            pltpu.VMEM_SHARED((NPAD,), jnp.float32),
            pltpu.SemaphoreType.DMA,
            pltpu.SemaphoreType.DMA,
            pltpu.SemaphoreType.DMA,
        ),
    )
    return run(feat, src3d, dst3d)


def _tc_finish(feat, acc, cnt, wt):
    R = 1000

    def body(feat_ref, acc_ref, cnt_ref, wt_ref, out_ref):
        agg = acc_ref[0] + acc_ref[1]
        c = cnt_ref[0] + cnt_ref[1]
        hn = jnp.where(c > 0.0, agg / jnp.maximum(c, 1.0), 0.0)
        x = (feat_ref[...] + hn) * 0.5
        out_ref[...] = jnp.maximum(
            jnp.dot(x, wt_ref[...], preferred_element_type=jnp.float32), 0.0)

    return pl.pallas_call(
        body,
        grid=(N_NODES // R,),
        in_specs=[
            pl.BlockSpec((R, D), lambda i: (i, 0)),
            pl.BlockSpec((NC, R, D), lambda i: (0, i, 0)),
            pl.BlockSpec((NC, R, 1), lambda i: (0, i, 0)),
            pl.BlockSpec((D, D), lambda i: (0, 0)),
        ],
        out_specs=pl.BlockSpec((R, D), lambda i: (i, 0)),
        out_shape=jax.ShapeDtypeStruct((N_NODES, D), jnp.float32),
    )(feat, acc, cnt, wt)


def kernel(feat, edge_index, W):
    n_fill = E_PAD - N_EDGES
    src = jnp.concatenate(
        [edge_index[0], jnp.zeros((n_fill,), jnp.int32)])
    trash = N_NODES + (jnp.arange(n_fill, dtype=jnp.int32) % (NPAD - N_NODES))
    dst = jnp.concatenate([edge_index[1], trash])
    acc, cnt = _sc_aggregate(feat, src.reshape(NW, NCHUNK, B),
                             dst.reshape(NW, NCHUNK, B))
    return _tc_finish(feat, acc, cnt.reshape(NC, NPAD, 1), W.T)


# speedup vs baseline: 2.3187x; 2.3187x over previous
import jax
import jax.numpy as jnp
from jax import lax
from jax.experimental import pallas as pl
from jax.experimental.pallas import tpu as pltpu
from jax.experimental.pallas import tpu_sc as plsc

N_NODES = 10000
N_EDGES = 320000
D = 128
NC = 2
NS = 16
NW = NC * NS
EPW = N_EDGES // NW
B = 80
NCHUNK = EPW // B
NPAD = 10240
RPT = NPAD // NS


def _sc_aggregate(feat, src3d, dst3d):
    mesh = plsc.VectorSubcoreMesh(
        core_axis_name="c", subcore_axis_name="s",
        num_cores=NC, num_subcores=NS)

    def body(feat_hbm, src_hbm, dst_hbm, acc_out, cnt_out,
             src_v, dst_v, rows_v, ones_v, zc_v,
             acc_sh, cnt_sh, gsem):
        cid = lax.axis_index("c")
        sid = lax.axis_index("s")
        wid = sid * NC + cid

        zv = jnp.zeros((16,), jnp.float32)
        ov = jnp.ones((16,), jnp.float32)

        def zrow_loop(i, carry):
            def zcol(k, c2):
                rows_v[i, pl.ds(k * 16, 16)] = zv
                return c2
            return lax.fori_loop(0, D // 16, zcol, carry)
        lax.fori_loop(0, B, zrow_loop, 0)

        def zc_loop(i, carry):
            zc_v[pl.ds(i * 16, 16)] = zv
            return carry
        lax.fori_loop(0, RPT // 16, zc_loop, 0)

        def o_loop(i, carry):
            ones_v[pl.ds(i * 16, 16)] = ov
            return carry
        lax.fori_loop(0, B // 16, o_loop, 0)

        base = sid * RPT
        for r in range(RPT // B):
            pltpu.sync_copy(rows_v, acc_sh.at[pl.ds(base + r * B, B)])
        pltpu.sync_copy(zc_v, cnt_sh.at[pl.ds(base, RPT)])

        pltpu.sync_copy(src_hbm.at[wid], src_v)
        pltpu.sync_copy(dst_hbm.at[wid], dst_v)

        plsc.subcore_barrier()

        def chunk(j, carry):
            pltpu.async_copy(feat_hbm.at[src_v.at[j]], rows_v, gsem).wait()
            pltpu.sync_copy(rows_v, acc_sh.at[dst_v.at[j]], add=True)
            pltpu.sync_copy(ones_v, cnt_sh.at[dst_v.at[j]], add=True)
            return carry
        lax.fori_loop(0, NCHUNK, chunk, 0)

        plsc.subcore_barrier()

        pltpu.sync_copy(acc_sh.at[pl.ds(base, RPT)],
                        acc_out.at[cid].at[pl.ds(base, RPT)])
        pltpu.sync_copy(cnt_sh.at[pl.ds(base, RPT)],
                        cnt_out.at[cid].at[pl.ds(base, RPT)])

    run = pl.kernel(
        body,
        out_type=(jax.ShapeDtypeStruct((NC, NPAD, D), jnp.float32),
                  jax.ShapeDtypeStruct((NC, NPAD), jnp.float32)),
        mesh=mesh,
        compiler_params=pltpu.CompilerParams(use_tc_tiling_on_sc=False),
        scratch_types=(
            pltpu.VMEM((NCHUNK, B), jnp.int32),
            pltpu.VMEM((NCHUNK, B), jnp.int32),
            pltpu.VMEM((B, D), jnp.float32),
            pltpu.VMEM((B,), jnp.float32),
            pltpu.VMEM((RPT,), jnp.float32),
            pltpu.VMEM_SHARED((NPAD, D), jnp.float32),
            pltpu.VMEM_SHARED((NPAD,), jnp.float32),
            pltpu.SemaphoreType.DMA,
        ),
    )
    return run(feat, src3d, dst3d)


def _tc_finish(feat, acc, cnt, wt):
    R = 1000

    def body(feat_ref, acc_ref, cnt_ref, wt_ref, out_ref):
        agg = acc_ref[0] + acc_ref[1]
        c = cnt_ref[0] + cnt_ref[1]
        hn = jnp.where(c > 0.0, agg / jnp.maximum(c, 1.0), 0.0)
        x = (feat_ref[...] + hn) * 0.5
        out_ref[...] = jnp.maximum(
            jnp.dot(x, wt_ref[...], preferred_element_type=jnp.float32), 0.0)

    return pl.pallas_call(
        body,
        grid=(N_NODES // R,),
        in_specs=[
            pl.BlockSpec((R, D), lambda i: (i, 0)),
            pl.BlockSpec((NC, R, D), lambda i: (0, i, 0)),
            pl.BlockSpec((NC, R, 1), lambda i: (0, i, 0)),
            pl.BlockSpec((D, D), lambda i: (0, 0)),
        ],
        out_specs=pl.BlockSpec((R, D), lambda i: (i, 0)),
        out_shape=jax.ShapeDtypeStruct((N_NODES, D), jnp.float32),
    )(feat, acc, cnt, wt)


def kernel(feat, edge_index, W):
    src3d = edge_index[0].reshape(NW, NCHUNK, B)
    dst3d = edge_index[1].reshape(NW, NCHUNK, B)
    acc, cnt = _sc_aggregate(feat, src3d, dst3d)
    return _tc_finish(feat, acc, cnt.reshape(NC, NPAD, 1), W.T)


# speedup vs baseline: 2.4079x; 1.0385x over previous
import jax
import jax.numpy as jnp
from jax import lax
from jax.experimental import pallas as pl
from jax.experimental.pallas import tpu as pltpu
from jax.experimental.pallas import tpu_sc as plsc

N_NODES = 10000
N_EDGES = 320000
D = 128
NC = 2
NS = 16
NW = NC * NS
EPW = N_EDGES // NW
B = 80
NCHUNK = EPW // B
NPAD = 10240
RPT = NPAD // NS


def _sc_aggregate(feat, src3d, dst3d):
    mesh = plsc.VectorSubcoreMesh(
        core_axis_name="c", subcore_axis_name="s",
        num_cores=NC, num_subcores=NS)

    def body(feat_hbm, src_hbm, dst_hbm, acc_out, cnt_out,
             src_v, dst_v, rows_v, ones_v, zc_v,
             acc_sh, cnt_sh, gsem, csem):
        cid = lax.axis_index("c")
        sid = lax.axis_index("s")
        wid = sid * NC + cid

        zv = jnp.zeros((16,), jnp.float32)
        ov = jnp.ones((16,), jnp.float32)

        def zrow_loop(i, carry):
            def zcol(k, c2):
                rows_v[i, pl.ds(k * 16, 16)] = zv
                return c2
            return lax.fori_loop(0, D // 16, zcol, carry)
        lax.fori_loop(0, B, zrow_loop, 0)

        def zc_loop(i, carry):
            zc_v[pl.ds(i * 16, 16)] = zv
            return carry
        lax.fori_loop(0, RPT // 16, zc_loop, 0)

        def o_loop(i, carry):
            ones_v[pl.ds(i * 16, 16)] = ov
            return carry
        lax.fori_loop(0, B // 16, o_loop, 0)

        base = sid * RPT
        for r in range(RPT // B):
            pltpu.sync_copy(rows_v, acc_sh.at[pl.ds(base + r * B, B)])
        pltpu.sync_copy(zc_v, cnt_sh.at[pl.ds(base, RPT)])

        pltpu.sync_copy(src_hbm.at[wid], src_v)
        pltpu.sync_copy(dst_hbm.at[wid], dst_v)

        plsc.subcore_barrier()

        def chunk(j, carry):
            pltpu.async_copy(feat_hbm.at[src_v.at[j]], rows_v, gsem).wait()
            pltpu.sync_copy(rows_v, acc_sh.at[dst_v.at[j]], add=True)
            pltpu.async_copy(ones_v, cnt_sh.at[dst_v.at[j]], csem, add=True)
            return carry
        lax.fori_loop(0, NCHUNK, chunk, 0)

        def drain(i, carry):
            pltpu.make_async_copy(
                ones_v, cnt_sh.at[pl.ds(0, B)], csem).wait()
            return carry
        lax.fori_loop(0, NCHUNK, drain, 0)

        plsc.subcore_barrier()

        pltpu.sync_copy(acc_sh.at[pl.ds(base, RPT)],
                        acc_out.at[cid].at[pl.ds(base, RPT)])
        pltpu.sync_copy(cnt_sh.at[pl.ds(base, RPT)],
                        cnt_out.at[cid].at[pl.ds(base, RPT)])

    run = pl.kernel(
        body,
        out_type=(jax.ShapeDtypeStruct((NC, NPAD, D), jnp.float32),
                  jax.ShapeDtypeStruct((NC, NPAD), jnp.float32)),
        mesh=mesh,
        compiler_params=pltpu.CompilerParams(use_tc_tiling_on_sc=False),
        scratch_types=(
            pltpu.VMEM((NCHUNK, B), jnp.int32),
            pltpu.VMEM((NCHUNK, B), jnp.int32),
            pltpu.VMEM((B, D), jnp.float32),
            pltpu.VMEM((B,), jnp.float32),
            pltpu.VMEM((RPT,), jnp.float32),
            pltpu.VMEM_SHARED((NPAD, D), jnp.float32),
            pltpu.VMEM_SHARED((NPAD,), jnp.float32),
            pltpu.SemaphoreType.DMA,
            pltpu.SemaphoreType.DMA,
        ),
    )
    return run(feat, src3d, dst3d)


def _tc_finish(feat, acc, cnt, wt):
    R = 1000

    def body(feat_ref, acc_ref, cnt_ref, wt_ref, out_ref):
        agg = acc_ref[0] + acc_ref[1]
        c = cnt_ref[0] + cnt_ref[1]
        hn = jnp.where(c > 0.0, agg / jnp.maximum(c, 1.0), 0.0)
        x = (feat_ref[...] + hn) * 0.5
        out_ref[...] = jnp.maximum(
            jnp.dot(x, wt_ref[...], preferred_element_type=jnp.float32), 0.0)

    return pl.pallas_call(
        body,
        grid=(N_NODES // R,),
        in_specs=[
            pl.BlockSpec((R, D), lambda i: (i, 0)),
            pl.BlockSpec((NC, R, D), lambda i: (0, i, 0)),
            pl.BlockSpec((NC, R, 1), lambda i: (0, i, 0)),
            pl.BlockSpec((D, D), lambda i: (0, 0)),
        ],
        out_specs=pl.BlockSpec((R, D), lambda i: (i, 0)),
        out_shape=jax.ShapeDtypeStruct((N_NODES, D), jnp.float32),
    )(feat, acc, cnt, wt)


def kernel(feat, edge_index, W):
    src3d = edge_index[0].reshape(NW, NCHUNK, B)
    dst3d = edge_index[1].reshape(NW, NCHUNK, B)
    acc, cnt = _sc_aggregate(feat, src3d, dst3d)
    return _tc_finish(feat, acc, cnt.reshape(NC, NPAD, 1), W.T)
